# Initial kernel scaffold; baseline (speedup 1.0000x reference)
import jax
import jax.numpy as jnp
from jax.experimental import pallas as pl


def kernel(x, edge_index, edge_w, W1, b1, W2, b2):
    raise NotImplementedError("write your pallas kernel here")


# speedup vs baseline: 4.7798x; 4.7798x over previous
import functools

import jax
import jax.numpy as jnp
from jax import lax
from jax.experimental import pallas as pl
from jax.experimental.pallas import tpu as pltpu
from jax.experimental.pallas import tpu_sc as plsc

N = 10000
E = 320000
D = 128
NCLS = 32

NC = 2
NS = 16
NW = NC * NS
EPW = E // NW
CHUNK = 80
NCHUNK = EPW // CHUNK
BLKC = 5
NBLK = NCHUNK // BLKC
NPAD = 10240
ROWS_PT = NPAD // NS
DEGW = 128
WROWS = 16


def _sc_agg(x, src, dst, w, z128):
    mesh = plsc.VectorSubcoreMesh(core_axis_name="c", subcore_axis_name="s")

    @functools.partial(
        pl.kernel,
        out_type=jax.ShapeDtypeStruct((NC, NPAD, D), jnp.float32),
        mesh=mesh,
        scratch_types=[
            pltpu.VMEM((EPW,), jnp.int32),
            pltpu.VMEM((BLKC, CHUNK), jnp.int32),
            pltpu.VMEM((WROWS, 128), jnp.float32),
            pltpu.VMEM((CHUNK, D), jnp.float32),
            pltpu.VMEM_SHARED((NPAD, D), jnp.float32),
            pltpu.SemaphoreType.DMA,
        ],
    )
    def k_agg(x_hbm, src_hbm, dst_hbm, w_hbm, z128_hbm, agg_out,
              src_v, dst_v, w_v, rows_v, agg_sh, sem):
        c = lax.axis_index("c")
        s = lax.axis_index("s")
        wid = c * NS + s
        base = s * ROWS_PT

        pltpu.sync_copy(z128_hbm.at[pl.ds(base, ROWS_PT)],
                        agg_sh.at[pl.ds(base, ROWS_PT)])
        pltpu.sync_copy(src_hbm.at[wid], src_v)

        plsc.subcore_barrier()

        def blk_body(jg, carry):
            pltpu.sync_copy(dst_hbm.at[wid, jg], dst_v)

            def chunk_body(jj, carry2):
                j = jg * BLKC + jj
                off = pl.multiple_of(j * CHUNK, CHUNK)
                gat = pltpu.async_copy(
                    x_hbm.at[src_v.at[pl.ds(off, CHUNK)]], rows_v, sem)
                pltpu.sync_copy(w_hbm.at[wid, j], w_v)
                gat.wait()

                def grp_body(i8, carry3):
                    for k in range(8):
                        wv = w_v[i8, pl.ds(k * 16, 16)]
                        r = i8 * 8 + k
                        for t in range(D // 16):
                            sl = pl.ds(t * 16, 16)
                            rows_v[r, sl] = rows_v[r, sl] * wv
                    return carry3

                lax.fori_loop(0, CHUNK // 8, grp_body, 0)

                pltpu.sync_copy(rows_v, agg_sh.at[dst_v.at[jj]], add=True)
                return carry2

            lax.fori_loop(0, BLKC, chunk_body, 0)
            return carry

        lax.fori_loop(0, NBLK, blk_body, 0)

        plsc.subcore_barrier()

        pltpu.sync_copy(agg_sh.at[pl.ds(base, ROWS_PT)],
                        agg_out.at[c, pl.ds(base, ROWS_PT)])

    return k_agg(x, src, dst, w, z128)


def _sc_deg(dst, z16, ones):
    mesh = plsc.VectorSubcoreMesh(core_axis_name="c", subcore_axis_name="s")

    @functools.partial(
        pl.kernel,
        out_type=jax.ShapeDtypeStruct((NC, NPAD, DEGW), jnp.float32),
        mesh=mesh,
        scratch_types=[
            pltpu.VMEM((BLKC, CHUNK), jnp.int32),
            pltpu.VMEM((CHUNK, DEGW), jnp.float32),
            pltpu.VMEM_SHARED((NPAD, DEGW), jnp.float32),
        ],
    )
    def k_deg(dst_hbm, z16_hbm, ones_hbm, deg_out, dst_v, ones_v, deg_sh):
        c = lax.axis_index("c")
        s = lax.axis_index("s")
        wid = c * NS + s
        base = s * ROWS_PT

        pltpu.sync_copy(z16_hbm.at[pl.ds(base, ROWS_PT)],
                        deg_sh.at[pl.ds(base, ROWS_PT)])
        pltpu.sync_copy(ones_hbm, ones_v)

        plsc.subcore_barrier()

        def blk_body(jg, carry):
            pltpu.sync_copy(dst_hbm.at[wid, jg], dst_v)

            def chunk_body(jj, carry2):
                pltpu.sync_copy(ones_v, deg_sh.at[dst_v.at[jj]], add=True)
                return carry2

            lax.fori_loop(0, BLKC, chunk_body, 0)
            return carry

        lax.fori_loop(0, NBLK, blk_body, 0)

        plsc.subcore_barrier()

        pltpu.sync_copy(deg_sh.at[pl.ds(base, ROWS_PT)],
                        deg_out.at[c, pl.ds(base, ROWS_PT)])

    return k_deg(dst, z16, ones)


def _tc_body(agg_ref, deg_ref, w1_ref, b1_ref, w2_ref, b2_ref, out_ref):
    agg = agg_ref[0] + agg_ref[1]
    deg = deg_ref[0, :, 0:1] + deg_ref[1, :, 0:1]
    h = jnp.tanh(agg / jnp.maximum(deg, 1.0))
    m = jnp.sum(h, axis=0, keepdims=True) * (1.0 / N)
    p = jnp.dot(m, w1_ref[...], preferred_element_type=jnp.float32) + b1_ref[...]
    out_ref[...] = (
        jnp.dot(p, w2_ref[...], preferred_element_type=jnp.float32) + b2_ref[...]
    )


def _tc_finish(agg_p, deg_p, W1, b1, W2, b2):
    return pl.pallas_call(
        _tc_body,
        out_shape=jax.ShapeDtypeStruct((1, NCLS), jnp.float32),
    )(agg_p, deg_p, W1, b1, W2, b2)


def kernel(x, edge_index, edge_w, W1, b1, W2, b2):
    src = edge_index[0].reshape(NW, EPW)
    dst = edge_index[1].reshape(NW, NBLK, BLKC, CHUNK)
    wb = jnp.broadcast_to(
        edge_w.reshape(NW, NCHUNK, CHUNK // 8, 8, 1),
        (NW, NCHUNK, CHUNK // 8, 8, 16),
    ).reshape(NW, NCHUNK, CHUNK * 16 // 128, 128)
    wb = jnp.pad(wb, ((0, 0), (0, 0), (0, WROWS - CHUNK * 16 // 128), (0, 0)))
    z128 = jnp.zeros((NPAD, D), jnp.float32)
    z16 = z128
    ones = jnp.ones((CHUNK, DEGW), jnp.float32)
    agg_p = _sc_agg(x, src, dst, wb, z128)
    deg_p = _sc_deg(dst, z16, ones)
    return _tc_finish(agg_p, deg_p, W1, b1.reshape(1, D), W2, b2.reshape(1, NCLS))


# speedup vs baseline: 6.3656x; 1.3318x over previous
import functools

import jax
import jax.numpy as jnp
from jax import lax
from jax.experimental import pallas as pl
from jax.experimental.pallas import tpu as pltpu
from jax.experimental.pallas import tpu_sc as plsc

N = 10000
E = 320000
D = 128
NCLS = 32

NC = 2
NS = 16
NW = NC * NS
EPW = E // NW
CHUNK = 80
NCHUNK = EPW // CHUNK
BLKC = 25
NBLK = NCHUNK // BLKC
NPAD = 10240
ROWS_PT = NPAD // NS
DEGW = 128
WROWS = CHUNK * 16 // 128


def _sc_segment(x, src, dst, w, z128):
    mesh = plsc.VectorSubcoreMesh(core_axis_name="c", subcore_axis_name="s")

    @functools.partial(
        pl.kernel,
        out_type=[
            jax.ShapeDtypeStruct((NC, NPAD, D), jnp.float32),
            jax.ShapeDtypeStruct((NC, NPAD, DEGW), jnp.float32),
        ],
        mesh=mesh,
        scratch_types=[
            pltpu.VMEM((BLKC, CHUNK), jnp.int32),
            pltpu.VMEM((BLKC, CHUNK), jnp.int32),
            pltpu.VMEM((WROWS, 128), jnp.float32),
            pltpu.VMEM((WROWS, 128), jnp.float32),
            pltpu.VMEM((CHUNK, D), jnp.float32),
            pltpu.VMEM((CHUNK, D), jnp.float32),
            pltpu.VMEM_SHARED((NPAD, D), jnp.float32),
            pltpu.SemaphoreType.DMA,
            pltpu.SemaphoreType.DMA,
            pltpu.SemaphoreType.DMA,
            pltpu.SemaphoreType.DMA,
            pltpu.SemaphoreType.DMA,
        ],
    )
    def k(x_hbm, src_hbm, dst_hbm, w_hbm, z128_hbm, agg_out, deg_out,
          src_v, dst_v, w_a, w_b, rows_a, rows_b, agg_sh,
          sem_ga, sem_gb, sem_wa, sem_wb, sem_s):
        c = lax.axis_index("c")
        s = lax.axis_index("s")
        wid = c * NS + s
        base = s * ROWS_PT

        rows = (rows_a, rows_b)
        wbuf = (w_a, w_b)
        gsem = (sem_ga, sem_gb)
        wsem = (sem_wa, sem_wb)

        def start_fetch(j, p):
            g = pltpu.async_copy(
                x_hbm.at[src_v.at[lax.rem(j, BLKC)]], rows[p], gsem[p])
            wcp = pltpu.async_copy(w_hbm.at[wid, j], wbuf[p], wsem[p])
            return g, wcp

        def scale_rows(p):
            def grp_body(i8, carry3):
                for k in range(8):
                    wv = wbuf[p][i8, pl.ds(k * 16, 16)]
                    r = i8 * 8 + k
                    for t in range(D // 16):
                        sl = pl.ds(t * 16, 16)
                        rows[p][r, sl] = rows[p][r, sl] * wv
                return carry3

            lax.fori_loop(0, CHUNK // 8, grp_body, 0)

        def process(j, p, last):
            @pl.when((lax.rem(j, BLKC) == 0) & (j > 0))
            def _():
                pltpu.sync_copy(dst_hbm.at[wid, j // BLKC], dst_v)

            descs = None
            if not last:
                jn = j + 1

                @pl.when(lax.rem(jn, BLKC) == 0)
                def _():
                    pltpu.sync_copy(src_hbm.at[wid, jn // BLKC], src_v)

                descs = start_fetch(jn, 1 - p)

            scale_rows(p)
            pltpu.sync_copy(
                rows[p], agg_sh.at[dst_v.at[lax.rem(j, BLKC)]], add=True)
            if descs is not None:
                descs[0].wait()
                descs[1].wait()

        pltpu.sync_copy(z128_hbm.at[pl.ds(base, ROWS_PT)],
                        agg_sh.at[pl.ds(base, ROWS_PT)])
        pltpu.sync_copy(src_hbm.at[wid, 0], src_v)
        pltpu.sync_copy(dst_hbm.at[wid, 0], dst_v)
        plsc.subcore_barrier()

        g0, w0 = start_fetch(0, 0)
        g0.wait()
        w0.wait()

        def pair_body(jp, carry):
            j0 = jp * 2
            process(j0, 0, last=False)
            process(j0 + 1, 1, last=False)
            return carry

        lax.fori_loop(0, (NCHUNK - 1) // 2, pair_body, 0)
        process(NCHUNK - 1, 0, last=True)

        plsc.subcore_barrier()
        pltpu.sync_copy(agg_sh.at[pl.ds(base, ROWS_PT)],
                        agg_out.at[c, pl.ds(base, ROWS_PT)])
        plsc.subcore_barrier()

        pltpu.sync_copy(z128_hbm.at[pl.ds(base, ROWS_PT)],
                        agg_sh.at[pl.ds(base, ROWS_PT)])

        onev = jnp.ones((16,), jnp.float32)

        def ones_body(r, carry):
            for t in range(D // 16):
                rows_a[r, pl.ds(t * 16, 16)] = onev
            return carry

        lax.fori_loop(0, CHUNK, ones_body, 0)
        plsc.subcore_barrier()

        def deg_blk(b, carry):
            pltpu.sync_copy(dst_hbm.at[wid, b], dst_v)

            def deg_chunk(jj, carry2):
                pltpu.sync_copy(rows_a, agg_sh.at[dst_v.at[jj]], add=True)
                return carry2

            lax.fori_loop(0, BLKC, deg_chunk, 0)
            return carry

        lax.fori_loop(0, NBLK, deg_blk, 0)

        plsc.subcore_barrier()
        pltpu.sync_copy(agg_sh.at[pl.ds(base, ROWS_PT)],
                        deg_out.at[c, pl.ds(base, ROWS_PT)])

    return k(x, src, dst, w, z128)


def _tc_body(agg_ref, deg_ref, w1_ref, b1_ref, w2_ref, b2_ref, out_ref):
    agg = agg_ref[0] + agg_ref[1]
    deg = deg_ref[0, :, 0:1] + deg_ref[1, :, 0:1]
    h = jnp.tanh(agg / jnp.maximum(deg, 1.0))
    m = jnp.sum(h, axis=0, keepdims=True) * (1.0 / N)
    p = jnp.dot(m, w1_ref[...], preferred_element_type=jnp.float32) + b1_ref[...]
    out_ref[...] = (
        jnp.dot(p, w2_ref[...], preferred_element_type=jnp.float32) + b2_ref[...]
    )


def _tc_finish(agg_p, deg_p, W1, b1, W2, b2):
    return pl.pallas_call(
        _tc_body,
        out_shape=jax.ShapeDtypeStruct((1, NCLS), jnp.float32),
    )(agg_p, deg_p, W1, b1, W2, b2)


def kernel(x, edge_index, edge_w, W1, b1, W2, b2):
    src = edge_index[0].reshape(NW, NBLK, BLKC, CHUNK)
    dst = edge_index[1].reshape(NW, NBLK, BLKC, CHUNK)
    wb = jnp.broadcast_to(
        edge_w.reshape(NW, NCHUNK, CHUNK // 8, 8, 1),
        (NW, NCHUNK, CHUNK // 8, 8, 16),
    ).reshape(NW, NCHUNK, WROWS, 128)
    z128 = jnp.zeros((NPAD, D), jnp.float32)
    agg_p, deg_p = _sc_segment(x, src, dst, wb, z128)
    return _tc_finish(agg_p, deg_p, W1, b1.reshape(1, D), W2, b2.reshape(1, NCLS))
